# baseline (speedup 1.0000x reference)
import functools

import jax
import jax.numpy as jnp
from jax import lax
from jax.experimental import pallas as pl
from jax.experimental.pallas import tpu as pltpu
from jax.experimental.pallas import tpu_sc as plsc

_NUM_MODELS = 2
_NUM_BOXES = 1000000
_DIM = 32
_BATCH = 16384

_NS = 16
_NPIECE = 244
_PBOX = 256
_WBOX = _NPIECE * _PBOX
_MAIN = _NS * _WBOX
_TAIL = _NUM_BOXES - _MAIN
_NTSUB = 4
_TSUB = _TAIL // _NTSUB
_ROW = 2 * _DIM
_CAP = 8192
_DRAIN = 4096

_mesh = plsc.VectorSubcoreMesh(core_axis_name="c", subcore_axis_name="s")


@functools.partial(
    pl.kernel,
    mesh=_mesh,
    compiler_params=pltpu.CompilerParams(needs_layout_passes=False),
    out_type=jax.ShapeDtypeStruct((_NUM_MODELS * _BATCH * _ROW,), jnp.float32),
    scratch_types=[
        pltpu.VMEM((512,), jnp.int32),
        pltpu.VMEM((_CAP + 32,), jnp.int32),
        pltpu.VMEM((_CAP + 32,), jnp.int32),
        pltpu.VMEM((_DIM, _PBOX), jnp.float32),
        pltpu.VMEM((_DIM, _PBOX), jnp.float32),
        pltpu.VMEM((_DIM, _PBOX), jnp.float32),
        pltpu.VMEM((_DIM, _PBOX), jnp.float32),
        pltpu.VMEM((_TSUB * _DIM,), jnp.float32),
        pltpu.VMEM((_TSUB * _DIM,), jnp.float32),
        pltpu.VMEM((16,), jnp.int32),
        pltpu.VMEM((16,), jnp.int32),
        pltpu.VMEM((_ROW,), jnp.float32),
        pltpu.VMEM((_DRAIN,), jnp.float32),
        pltpu.VMEM_SHARED((_BATCH * _ROW,), jnp.float32),
        pltpu.SemaphoreType.DMA,
        pltpu.SemaphoreType.DMA,
    ],
)
def _deltaboxes_sc(ids_hbm, zt, ldt, ztail, ldtail, out_hbm,
                   idw, sel_id, sel_pos, zst0, ldst0, zst1, ldst1,
                   tailz, taill, gid, gpos, rowb, dbounce, shared,
                   semA, semB):
    c = lax.axis_index("c")
    s = lax.axis_index("s")
    lo = s * _WBOX
    hi = jnp.where(s == _NS - 1, _NUM_BOXES, lo + _WBOX)
    lane = lax.iota(jnp.int32, 16)

    def do_round(p0):

        def sel_chunk(ch, st):
            pltpu.sync_copy(ids_hbm.at[pl.ds(ch * 512, 512)], idw)

            def sel_group(q, st):
                nsel, pnext = st
                idv = idw[pl.ds(q * 16, 16)]
                gstart = ch * 512 + q * 16
                pos = lane + gstart
                ok = nsel <= _CAP - 16
                mask = (idv >= lo) & (idv < hi) & (pos >= p0) & ok
                plsc.store_compressed(
                    sel_id.at[pl.ds(nsel, 16)], idv, mask=mask)
                plsc.store_compressed(
                    sel_pos.at[pl.ds(nsel, 16)], pos, mask=mask)
                nsel = nsel + plsc.all_reduce_population_count(mask)[0]
                pnext = jnp.where(ok, pnext, jnp.minimum(pnext, gstart))
                return (nsel, pnext)

            return lax.fori_loop(0, 32, sel_group, st)

        nsel, pnext = lax.fori_loop(0, 32, sel_chunk, (0, _BATCH))
        big = jnp.full((16,), jnp.int32(0x40000000))
        sel_id[pl.ds(nsel, 16)] = big
        sel_id[pl.ds(nsel + 16, 16)] = big
        ngroups = (nsel + 15) // 16

        def emit_rows(plo, phi, extract):
            def scan_group(g, carry):
                sid = sel_id[pl.ds(g * 16, 16)]
                mask = (sid >= plo) & (sid < phi)
                cnt = plsc.all_reduce_population_count(mask)[0]

                @pl.when(cnt > 0)
                def _():
                    spos = sel_pos[pl.ds(g * 16, 16)]
                    plsc.store_compressed(gid.at[pl.ds(0, 16)], sid, mask=mask)
                    plsc.store_compressed(
                        gpos.at[pl.ds(0, 16)], spos, mask=mask)
                    shift = jnp.minimum(lane + 1, 15)

                    def one(t, carry2):
                        gv, pv = carry2
                        bl = gv[0] - plo
                        for k in range(2):
                            dvec = lane + (k * 16)
                            zs, ls = extract(bl, dvec)
                            rowb[pl.ds(k * 16, 16)] = zs
                            rowb[pl.ds(_DIM + k * 16, 16)] = zs + jnp.exp(ls)
                        pltpu.sync_copy(
                            rowb, shared.at[pl.ds(pv[0] * _ROW, _ROW)])
                        return (gv.at[shift].get(mode="promise_in_bounds"),
                                pv.at[shift].get(mode="promise_in_bounds"))

                    lax.fori_loop(0, cnt, one,
                                  (gid[pl.ds(0, 16)], gpos[pl.ds(0, 16)]))

                return carry

            lax.fori_loop(0, ngroups, scan_group, 0)

        def fire(plo, zb, lb, sem):
            pltpu.async_copy(
                zt.at[c, pl.ds(0, _DIM), pl.ds(plo, _PBOX)], zb, sem)
            pltpu.async_copy(
                ldt.at[c, pl.ds(0, _DIM), pl.ds(plo, _PBOX)], lb, sem)

        def wait_piece(zb, lb, sem):
            pltpu.make_async_copy(
                zt.at[c, pl.ds(0, _DIM), pl.ds(0, _PBOX)], zb, sem).wait()
            pltpu.make_async_copy(
                ldt.at[c, pl.ds(0, _DIM), pl.ds(0, _PBOX)], lb, sem).wait()

        def process(zb, lb, plo):
            def extract(bl, dvec):
                bvec = jnp.full((16,), 0, jnp.int32) + bl
                return (plsc.load_gather(zb, [dvec, bvec]),
                        plsc.load_gather(lb, [dvec, bvec]))

            emit_rows(plo, plo + _PBOX, extract)

        fire(lo, zst0, ldst0, semA)

        def piece2(p2, carry):
            plo = lo + p2 * (2 * _PBOX)
            fire(plo + _PBOX, zst1, ldst1, semB)
            wait_piece(zst0, ldst0, semA)
            process(zst0, ldst0, plo)

            @pl.when(p2 < _NPIECE // 2 - 1)
            def _():
                fire(plo + 2 * _PBOX, zst0, ldst0, semA)

            wait_piece(zst1, ldst1, semB)
            process(zst1, ldst1, plo + _PBOX)
            return carry

        lax.fori_loop(0, _NPIECE // 2, piece2, 0)

        @pl.when(s == _NS - 1)
        def _():
            for tp in range(_NTSUB):
                toff = c * (_TAIL * _DIM) + tp * (_TSUB * _DIM)
                pltpu.sync_copy(ztail.at[pl.ds(toff, _TSUB * _DIM)], tailz)
                pltpu.sync_copy(ldtail.at[pl.ds(toff, _TSUB * _DIM)], taill)
                tlo = _MAIN + tp * _TSUB

                def extract(bl, dvec):
                    idx = jnp.full((16,), 0, jnp.int32) + bl * _DIM + dvec
                    return (plsc.load_gather(tailz, [idx]),
                            plsc.load_gather(taill, [idx]))

                emit_rows(tlo, tlo + _TSUB, extract)

        return pnext

    p1 = do_round(0)

    @pl.when(p1 < _BATCH)
    def _():
        do_round(p1)

    plsc.subcore_barrier()
    span = _BATCH * _ROW // _NS

    def drain(d, carry):
        off = s * span + d * _DRAIN
        pltpu.sync_copy(shared.at[pl.ds(off, _DRAIN)], dbounce)
        pltpu.sync_copy(
            dbounce, out_hbm.at[pl.ds(c * (_BATCH * _ROW) + off, _DRAIN)])
        return carry

    lax.fori_loop(0, span // _DRAIN, drain, 0)


def kernel(ids, z, logdelta):
    zt = jnp.transpose(z, (0, 2, 1))
    ldt = jnp.transpose(logdelta, (0, 2, 1))
    ztail = z[:, _MAIN:, :].reshape(-1)
    ldtail = logdelta[:, _MAIN:, :].reshape(-1)
    flat = _deltaboxes_sc(ids.astype(jnp.int32), zt, ldt, ztail, ldtail)
    return flat.reshape(_NUM_MODELS, _BATCH, 2, _DIM)


# speedup vs baseline: 1.4519x; 1.4519x over previous
import functools

import jax
import jax.numpy as jnp
from jax import lax
from jax.experimental import pallas as pl
from jax.experimental.pallas import tpu as pltpu
from jax.experimental.pallas import tpu_sc as plsc

_NUM_MODELS = 2
_NUM_BOXES = 1000000
_DIM = 32
_BATCH = 16384

_NS = 16
_NPIECE = 244
_PBOX = 256
_WBOX = _NPIECE * _PBOX
_MAIN = _NS * _WBOX
_TAIL = _NUM_BOXES - _MAIN
_NTSUB = 8
_TSUB = _TAIL // _NTSUB
_ROW = 2 * _DIM
_CAP = 8192
_SCAP = 2048
_DRAIN = 1024

_mesh = plsc.VectorSubcoreMesh(core_axis_name="c", subcore_axis_name="s")


@functools.partial(
    pl.kernel,
    mesh=_mesh,
    compiler_params=pltpu.CompilerParams(needs_layout_passes=False),
    out_type=jax.ShapeDtypeStruct((_NUM_MODELS * _BATCH * _ROW,), jnp.float32),
    scratch_types=[
        pltpu.VMEM((512,), jnp.int32),
        pltpu.VMEM((_CAP + 32,), jnp.int32),
        pltpu.VMEM((_CAP + 32,), jnp.int32),
        pltpu.VMEM((_DIM, _PBOX), jnp.float32),
        pltpu.VMEM((_DIM, _PBOX), jnp.float32),
        pltpu.VMEM((_DIM, _PBOX), jnp.float32),
        pltpu.VMEM((_DIM, _PBOX), jnp.float32),
        pltpu.VMEM((_TSUB * _DIM,), jnp.float32),
        pltpu.VMEM((_TSUB * _DIM,), jnp.float32),
        pltpu.VMEM((_SCAP + 32,), jnp.int32),
        pltpu.VMEM((_SCAP + 32,), jnp.int32),
        pltpu.VMEM((16,), jnp.int32),
        pltpu.VMEM((16,), jnp.int32),
        pltpu.VMEM((_ROW,), jnp.float32),
        pltpu.VMEM((_DRAIN,), jnp.float32),
        pltpu.VMEM_SHARED((_BATCH * _ROW,), jnp.float32),
        pltpu.SemaphoreType.DMA,
        pltpu.SemaphoreType.DMA,
    ],
)
def _deltaboxes_sc(ids_hbm, zt, ldt, ztail, ldtail, out_hbm,
                   idw, sel_id, sel_pos, zst0, ldst0, zst1, ldst1,
                   tailz, taill, sub_id, sub_pos, gid, gpos, rowb,
                   dbounce, shared, semA, semB):
    c = lax.axis_index("c")
    s = lax.axis_index("s")
    lo = s * _WBOX
    hi = jnp.where(s == _NS - 1, _NUM_BOXES, lo + _WBOX)
    lane = lax.iota(jnp.int32, 16)

    def do_round(p0):

        def sel_chunk(ch, st):
            pltpu.sync_copy(ids_hbm.at[pl.ds(ch * 512, 512)], idw)

            def sel_group(q, st):
                nsel, pnext = st
                idv = idw[pl.ds(q * 16, 16)]
                gstart = ch * 512 + q * 16
                pos = lane + gstart
                ok = nsel <= _CAP - 16
                mask = (idv >= lo) & (idv < hi) & (pos >= p0) & ok
                plsc.store_compressed(
                    sel_id.at[pl.ds(nsel, 16)], idv, mask=mask)
                plsc.store_compressed(
                    sel_pos.at[pl.ds(nsel, 16)], pos, mask=mask)
                nsel = nsel + plsc.all_reduce_population_count(mask)[0]
                pnext = jnp.where(ok, pnext, jnp.minimum(pnext, gstart))
                return (nsel, pnext)

            return lax.fori_loop(0, 32, sel_group, st)

        nsel, pnext = lax.fori_loop(0, 32, sel_chunk, (0, _BATCH))
        big = jnp.full((16,), jnp.int32(0x40000000))
        sel_id[pl.ds(nsel, 16)] = big
        sel_id[pl.ds(nsel + 16, 16)] = big
        ngroups = (nsel + 15) // 16

        def emit_from(ids_ref, pos_ref, ng, plo, phi, extract):
            def scan_group(g, carry):
                sid = ids_ref[pl.ds(g * 16, 16)]
                mask = (sid >= plo) & (sid < phi)
                cnt = plsc.all_reduce_population_count(mask)[0]

                @pl.when(cnt > 0)
                def _():
                    spos = pos_ref[pl.ds(g * 16, 16)]
                    plsc.store_compressed(gid.at[pl.ds(0, 16)], sid, mask=mask)
                    plsc.store_compressed(
                        gpos.at[pl.ds(0, 16)], spos, mask=mask)
                    shift = jnp.minimum(lane + 1, 15)

                    def one(t, carry2):
                        gv, pv = carry2
                        bl = gv[0] - plo
                        for k in range(2):
                            dvec = lane + (k * 16)
                            zs, ls = extract(bl, dvec)
                            rowb[pl.ds(k * 16, 16)] = zs
                            rowb[pl.ds(_DIM + k * 16, 16)] = zs + jnp.exp(ls)
                        pltpu.sync_copy(
                            rowb, shared.at[pl.ds(pv[0] * _ROW, _ROW)])
                        return (gv.at[shift].get(mode="promise_in_bounds"),
                                pv.at[shift].get(mode="promise_in_bounds"))

                    lax.fori_loop(0, cnt, one,
                                  (gid[pl.ds(0, 16)], gpos[pl.ds(0, 16)]))

                return carry

            lax.fori_loop(0, ng, scan_group, 0)

        def build_sub(plo):
            def bg(g, cnt):
                sid = sel_id[pl.ds(g * 16, 16)]
                spos = sel_pos[pl.ds(g * 16, 16)]
                ok = cnt <= _SCAP - 16
                m = (sid >= plo) & (sid < plo + 2 * _PBOX) & ok
                plsc.store_compressed(sub_id.at[pl.ds(cnt, 16)], sid, mask=m)
                plsc.store_compressed(
                    sub_pos.at[pl.ds(cnt, 16)], spos, mask=m)
                return cnt + plsc.all_reduce_population_count(m)[0]

            cnt = lax.fori_loop(0, ngroups, bg, 0)
            big = jnp.full((16,), jnp.int32(0x40000000))
            sub_id[pl.ds(cnt, 16)] = big
            sub_id[pl.ds(cnt + 16, 16)] = big
            return cnt

        def emit_rows_sub(subcnt, plo, phi, extract):
            subok = subcnt <= _SCAP - 16

            @pl.when(subok)
            def _():
                emit_from(sub_id, sub_pos, (subcnt + 15) // 16,
                          plo, phi, extract)

            @pl.when(jnp.logical_not(subok))
            def _():
                emit_from(sel_id, sel_pos, ngroups, plo, phi, extract)

        def fire(plo, zb, lb, sem):
            pltpu.async_copy(
                zt.at[c, pl.ds(0, _DIM), pl.ds(plo, _PBOX)], zb, sem)
            pltpu.async_copy(
                ldt.at[c, pl.ds(0, _DIM), pl.ds(plo, _PBOX)], lb, sem)

        def wait_piece(zb, lb, sem):
            pltpu.make_async_copy(
                zt.at[c, pl.ds(0, _DIM), pl.ds(0, _PBOX)], zb, sem).wait()
            pltpu.make_async_copy(
                ldt.at[c, pl.ds(0, _DIM), pl.ds(0, _PBOX)], lb, sem).wait()

        def process(zb, lb, plo, subcnt):
            def extract(bl, dvec):
                bvec = jnp.full((16,), 0, jnp.int32) + bl
                return (plsc.load_gather(zb, [dvec, bvec]),
                        plsc.load_gather(lb, [dvec, bvec]))

            emit_rows_sub(subcnt, plo, plo + _PBOX, extract)

        fire(lo, zst0, ldst0, semA)

        def piece2(p2, carry):
            plo = lo + p2 * (2 * _PBOX)
            fire(plo + _PBOX, zst1, ldst1, semB)
            subcnt = build_sub(plo)
            wait_piece(zst0, ldst0, semA)
            process(zst0, ldst0, plo, subcnt)

            @pl.when(p2 < _NPIECE // 2 - 1)
            def _():
                fire(plo + 2 * _PBOX, zst0, ldst0, semA)

            wait_piece(zst1, ldst1, semB)
            process(zst1, ldst1, plo + _PBOX, subcnt)
            return carry

        lax.fori_loop(0, _NPIECE // 2, piece2, 0)

        @pl.when(s == _NS - 1)
        def _():
            for tp in range(_NTSUB):
                toff = c * (_TAIL * _DIM) + tp * (_TSUB * _DIM)
                pltpu.sync_copy(ztail.at[pl.ds(toff, _TSUB * _DIM)], tailz)
                pltpu.sync_copy(ldtail.at[pl.ds(toff, _TSUB * _DIM)], taill)
                tlo = _MAIN + tp * _TSUB

                def extract(bl, dvec):
                    idx = jnp.full((16,), 0, jnp.int32) + bl * _DIM + dvec
                    return (plsc.load_gather(tailz, [idx]),
                            plsc.load_gather(taill, [idx]))

                emit_from(sel_id, sel_pos, ngroups, tlo, tlo + _TSUB, extract)

        return pnext

    p1 = do_round(0)

    @pl.when(p1 < _BATCH)
    def _():
        do_round(p1)

    plsc.subcore_barrier()
    span = _BATCH * _ROW // _NS

    def drain(d, carry):
        off = s * span + d * _DRAIN
        pltpu.sync_copy(shared.at[pl.ds(off, _DRAIN)], dbounce)
        pltpu.sync_copy(
            dbounce, out_hbm.at[pl.ds(c * (_BATCH * _ROW) + off, _DRAIN)])
        return carry

    lax.fori_loop(0, span // _DRAIN, drain, 0)


def kernel(ids, z, logdelta):
    zt = jnp.transpose(z, (0, 2, 1))
    ldt = jnp.transpose(logdelta, (0, 2, 1))
    ztail = z[:, _MAIN:, :].reshape(-1)
    ldtail = logdelta[:, _MAIN:, :].reshape(-1)
    flat = _deltaboxes_sc(ids.astype(jnp.int32), zt, ldt, ztail, ldtail)
    return flat.reshape(_NUM_MODELS, _BATCH, 2, _DIM)


# speedup vs baseline: 1.5163x; 1.0443x over previous
import functools

import jax
import jax.numpy as jnp
from jax import lax
from jax.experimental import pallas as pl
from jax.experimental.pallas import tpu as pltpu
from jax.experimental.pallas import tpu_sc as plsc

_NUM_MODELS = 2
_NUM_BOXES = 1000000
_DIM = 32
_BATCH = 16384

_NS = 16
_NPIECE = 244
_PBOX = 256
_WBOX = _NPIECE * _PBOX
_MAIN = _NS * _WBOX
_TAIL = _NUM_BOXES - _MAIN
_NTSUB = 8
_TSUB = _TAIL // _NTSUB
_ROW = 2 * _DIM
_CAP = 8192
_SCAP = 2048
_DRAIN = 1024

_mesh = plsc.VectorSubcoreMesh(core_axis_name="c", subcore_axis_name="s")


@functools.partial(
    pl.kernel,
    mesh=_mesh,
    compiler_params=pltpu.CompilerParams(needs_layout_passes=False),
    out_type=jax.ShapeDtypeStruct((_NUM_MODELS * _BATCH * _ROW,), jnp.float32),
    scratch_types=[
        pltpu.VMEM((512,), jnp.int32),
        pltpu.VMEM((512,), jnp.int32),
        pltpu.VMEM((_CAP + 32,), jnp.int32),
        pltpu.VMEM((_CAP + 32,), jnp.int32),
        pltpu.VMEM((_DIM, _PBOX), jnp.float32),
        pltpu.VMEM((_DIM, _PBOX), jnp.float32),
        pltpu.VMEM((_DIM, _PBOX), jnp.float32),
        pltpu.VMEM((_DIM, _PBOX), jnp.float32),
        pltpu.VMEM((_TSUB * _DIM,), jnp.float32),
        pltpu.VMEM((_TSUB * _DIM,), jnp.float32),
        pltpu.VMEM((_SCAP + 32,), jnp.int32),
        pltpu.VMEM((_SCAP + 32,), jnp.int32),
        pltpu.VMEM((16,), jnp.int32),
        pltpu.VMEM((16,), jnp.int32),
        pltpu.VMEM((_ROW,), jnp.float32),
        pltpu.VMEM((_DRAIN,), jnp.float32),
        pltpu.VMEM((_DRAIN,), jnp.float32),
        pltpu.VMEM_SHARED((_BATCH * _ROW,), jnp.float32),
        pltpu.SemaphoreType.DMA,
        pltpu.SemaphoreType.DMA,
    ],
)
def _deltaboxes_sc(ids_hbm, zt, ldt, ztail, ldtail, out_hbm,
                   idw0, idw1, sel_id, sel_pos, zst0, ldst0, zst1, ldst1,
                   tailz, taill, sub_id, sub_pos, gid, gpos, rowb,
                   dbounce0, dbounce1, shared, semA, semB):
    c = lax.axis_index("c")
    s = lax.axis_index("s")
    lo = s * _WBOX
    hi = jnp.where(s == _NS - 1, _NUM_BOXES, lo + _WBOX)
    lane = lax.iota(jnp.int32, 16)

    def do_round(p0):

        def scan_ids(buf, base, st):
            def sel_group(q, st):
                nsel, pnext = st
                idv = buf[pl.ds(q * 16, 16)]
                gstart = base + q * 16
                pos = lane + gstart
                ok = nsel <= _CAP - 16
                mask = (idv >= lo) & (idv < hi) & (pos >= p0) & ok
                plsc.store_compressed(
                    sel_id.at[pl.ds(nsel, 16)], idv, mask=mask)
                plsc.store_compressed(
                    sel_pos.at[pl.ds(nsel, 16)], pos, mask=mask)
                nsel = nsel + plsc.all_reduce_population_count(mask)[0]
                pnext = jnp.where(ok, pnext, jnp.minimum(pnext, gstart))
                return (nsel, pnext)

            return lax.fori_loop(0, 32, sel_group, st)

        def wait_ids(buf, sem):
            pltpu.make_async_copy(
                ids_hbm.at[pl.ds(0, 512)], buf, sem).wait()

        pltpu.async_copy(ids_hbm.at[pl.ds(0, 512)], idw0, semA)

        def sel_pair(cp, st):
            base = cp * 1024
            pltpu.async_copy(
                ids_hbm.at[pl.ds(base + 512, 512)], idw1, semB)
            wait_ids(idw0, semA)
            st = scan_ids(idw0, base, st)

            @pl.when(cp < 15)
            def _():
                pltpu.async_copy(
                    ids_hbm.at[pl.ds(base + 1024, 512)], idw0, semA)

            wait_ids(idw1, semB)
            return scan_ids(idw1, base + 512, st)

        nsel, pnext = lax.fori_loop(0, 16, sel_pair, (0, _BATCH))
        big = jnp.full((16,), jnp.int32(0x40000000))
        sel_id[pl.ds(nsel, 16)] = big
        sel_id[pl.ds(nsel + 16, 16)] = big
        ngroups = (nsel + 15) // 16

        def emit_from(ids_ref, pos_ref, ng, plo, phi, extract):
            def scan_group(g, carry):
                sid = ids_ref[pl.ds(g * 16, 16)]
                mask = (sid >= plo) & (sid < phi)
                cnt = plsc.all_reduce_population_count(mask)[0]

                @pl.when(cnt > 0)
                def _():
                    spos = pos_ref[pl.ds(g * 16, 16)]
                    plsc.store_compressed(gid.at[pl.ds(0, 16)], sid, mask=mask)
                    plsc.store_compressed(
                        gpos.at[pl.ds(0, 16)], spos, mask=mask)
                    shift = jnp.minimum(lane + 1, 15)

                    def one(t, carry2):
                        gv, pv = carry2
                        bl = gv[0] - plo
                        for k in range(2):
                            dvec = lane + (k * 16)
                            zs, ls = extract(bl, dvec)
                            rowb[pl.ds(k * 16, 16)] = zs
                            rowb[pl.ds(_DIM + k * 16, 16)] = zs + jnp.exp(ls)
                        pltpu.sync_copy(
                            rowb, shared.at[pl.ds(pv[0] * _ROW, _ROW)])
                        return (gv.at[shift].get(mode="promise_in_bounds"),
                                pv.at[shift].get(mode="promise_in_bounds"))

                    lax.fori_loop(0, cnt, one,
                                  (gid[pl.ds(0, 16)], gpos[pl.ds(0, 16)]))

                return carry

            lax.fori_loop(0, ng, scan_group, 0)

        def build_sub(plo):
            def bg(g, cnt):
                sid = sel_id[pl.ds(g * 16, 16)]
                spos = sel_pos[pl.ds(g * 16, 16)]
                ok = cnt <= _SCAP - 16
                m = (sid >= plo) & (sid < plo + 2 * _PBOX) & ok
                plsc.store_compressed(sub_id.at[pl.ds(cnt, 16)], sid, mask=m)
                plsc.store_compressed(
                    sub_pos.at[pl.ds(cnt, 16)], spos, mask=m)
                return cnt + plsc.all_reduce_population_count(m)[0]

            cnt = lax.fori_loop(0, ngroups, bg, 0)
            big = jnp.full((16,), jnp.int32(0x40000000))
            sub_id[pl.ds(cnt, 16)] = big
            sub_id[pl.ds(cnt + 16, 16)] = big
            return cnt

        def emit_rows_sub(subcnt, plo, phi, extract):
            subok = subcnt <= _SCAP - 16

            @pl.when(subok)
            def _():
                emit_from(sub_id, sub_pos, (subcnt + 15) // 16,
                          plo, phi, extract)

            @pl.when(jnp.logical_not(subok))
            def _():
                emit_from(sel_id, sel_pos, ngroups, plo, phi, extract)

        def fire(plo, zb, lb, sem):
            pltpu.async_copy(
                zt.at[c, pl.ds(0, _DIM), pl.ds(plo, _PBOX)], zb, sem)
            pltpu.async_copy(
                ldt.at[c, pl.ds(0, _DIM), pl.ds(plo, _PBOX)], lb, sem)

        def wait_piece(zb, lb, sem):
            pltpu.make_async_copy(
                zt.at[c, pl.ds(0, _DIM), pl.ds(0, _PBOX)], zb, sem).wait()
            pltpu.make_async_copy(
                ldt.at[c, pl.ds(0, _DIM), pl.ds(0, _PBOX)], lb, sem).wait()

        def process(zb, lb, plo, subcnt):
            def extract(bl, dvec):
                bvec = jnp.full((16,), 0, jnp.int32) + bl
                return (plsc.load_gather(zb, [dvec, bvec]),
                        plsc.load_gather(lb, [dvec, bvec]))

            emit_rows_sub(subcnt, plo, plo + _PBOX, extract)

        fire(lo, zst0, ldst0, semA)

        def piece2(p2, carry):
            plo = lo + p2 * (2 * _PBOX)
            fire(plo + _PBOX, zst1, ldst1, semB)
            subcnt = build_sub(plo)
            wait_piece(zst0, ldst0, semA)
            process(zst0, ldst0, plo, subcnt)

            @pl.when(p2 < _NPIECE // 2 - 1)
            def _():
                fire(plo + 2 * _PBOX, zst0, ldst0, semA)

            wait_piece(zst1, ldst1, semB)
            process(zst1, ldst1, plo + _PBOX, subcnt)
            return carry

        lax.fori_loop(0, _NPIECE // 2, piece2, 0)

        @pl.when(s == _NS - 1)
        def _():
            for tp in range(_NTSUB):
                toff = c * (_TAIL * _DIM) + tp * (_TSUB * _DIM)
                pltpu.sync_copy(ztail.at[pl.ds(toff, _TSUB * _DIM)], tailz)
                pltpu.sync_copy(ldtail.at[pl.ds(toff, _TSUB * _DIM)], taill)
                tlo = _MAIN + tp * _TSUB

                def extract(bl, dvec):
                    idx = jnp.full((16,), 0, jnp.int32) + bl * _DIM + dvec
                    return (plsc.load_gather(tailz, [idx]),
                            plsc.load_gather(taill, [idx]))

                emit_from(sel_id, sel_pos, ngroups, tlo, tlo + _TSUB, extract)

        return pnext

    p1 = do_round(0)

    @pl.when(p1 < _BATCH)
    def _():
        do_round(p1)

    plsc.subcore_barrier()
    span = _BATCH * _ROW // _NS
    obase = c * (_BATCH * _ROW)

    def wait_out(buf, sem):
        pltpu.make_async_copy(buf, out_hbm.at[pl.ds(0, _DRAIN)], sem).wait()

    def drain_pair(d, carry):
        off = s * span + d * (2 * _DRAIN)

        @pl.when(d > 0)
        def _():
            wait_out(dbounce0, semA)

        pltpu.sync_copy(shared.at[pl.ds(off, _DRAIN)], dbounce0)
        pltpu.async_copy(dbounce0, out_hbm.at[pl.ds(obase + off, _DRAIN)],
                         semA)

        @pl.when(d > 0)
        def _():
            wait_out(dbounce1, semB)

        pltpu.sync_copy(shared.at[pl.ds(off + _DRAIN, _DRAIN)], dbounce1)
        pltpu.async_copy(
            dbounce1, out_hbm.at[pl.ds(obase + off + _DRAIN, _DRAIN)], semB)
        return carry

    lax.fori_loop(0, span // (2 * _DRAIN), drain_pair, 0)
    wait_out(dbounce0, semA)
    wait_out(dbounce1, semB)


def kernel(ids, z, logdelta):
    zt = jnp.transpose(z, (0, 2, 1))
    ldt = jnp.transpose(logdelta, (0, 2, 1))
    ztail = z[:, _MAIN:, :].reshape(-1)
    ldtail = logdelta[:, _MAIN:, :].reshape(-1)
    flat = _deltaboxes_sc(ids.astype(jnp.int32), zt, ldt, ztail, ldtail)
    return flat.reshape(_NUM_MODELS, _BATCH, 2, _DIM)


# speedup vs baseline: 1.5277x; 1.0075x over previous
import functools

import jax
import jax.numpy as jnp
from jax import lax
from jax.experimental import pallas as pl
from jax.experimental.pallas import tpu as pltpu
from jax.experimental.pallas import tpu_sc as plsc

_NUM_MODELS = 2
_NUM_BOXES = 1000000
_DIM = 32
_BATCH = 16384

_NS = 16
_NPIECE = 244
_PBOX = 256
_WBOX = _NPIECE * _PBOX
_MAIN = _NS * _WBOX
_TAIL = _NUM_BOXES - _MAIN
_NTSUB = 8
_TSUB = _TAIL // _NTSUB
_ROW = 2 * _DIM
_CAP = 8192
_SCAP = 2048
_DRAIN = 2048

_mesh = plsc.VectorSubcoreMesh(core_axis_name="c", subcore_axis_name="s")


@functools.partial(
    pl.kernel,
    mesh=_mesh,
    compiler_params=pltpu.CompilerParams(needs_layout_passes=False),
    out_type=jax.ShapeDtypeStruct((_NUM_MODELS * _BATCH * _ROW,), jnp.float32),
    scratch_types=[
        pltpu.VMEM((512,), jnp.int32),
        pltpu.VMEM((512,), jnp.int32),
        pltpu.VMEM((_CAP + 32,), jnp.int32),
        pltpu.VMEM((_CAP + 32,), jnp.int32),
        pltpu.VMEM((_DIM, _PBOX), jnp.float32),
        pltpu.VMEM((_DIM, _PBOX), jnp.float32),
        pltpu.VMEM((_DIM, _PBOX), jnp.float32),
        pltpu.VMEM((_DIM, _PBOX), jnp.float32),
        pltpu.VMEM((_TSUB * _DIM,), jnp.float32),
        pltpu.VMEM((_TSUB * _DIM,), jnp.float32),
        pltpu.VMEM((_SCAP + 32,), jnp.int32),
        pltpu.VMEM((_SCAP + 32,), jnp.int32),
        pltpu.VMEM((16,), jnp.int32),
        pltpu.VMEM((16,), jnp.int32),
        pltpu.VMEM((_ROW,), jnp.float32),
        pltpu.VMEM((_DRAIN,), jnp.float32),
        pltpu.VMEM((_DRAIN,), jnp.float32),
        pltpu.VMEM_SHARED((_BATCH * _ROW,), jnp.float32),
        pltpu.SemaphoreType.DMA,
        pltpu.SemaphoreType.DMA,
    ],
)
def _deltaboxes_sc(ids_hbm, zt, ldt, ztail, ldtail, out_hbm,
                   idw0, idw1, sel_id, sel_pos, zst0, ldst0, zst1, ldst1,
                   tailz, taill, sub_id, sub_pos, gid, gpos, rowb,
                   dbounce0, dbounce1, shared, semA, semB):
    c = lax.axis_index("c")
    s = lax.axis_index("s")
    lo = s * _WBOX
    hi = jnp.where(s == _NS - 1, _NUM_BOXES, lo + _WBOX)
    lane = lax.iota(jnp.int32, 16)

    def do_round(p0):

        def scan_ids(buf, base, st):
            def sel_group(q, st):
                nsel, pnext = st
                idv = buf[pl.ds(q * 16, 16)]
                gstart = base + q * 16
                pos = lane + gstart
                ok = nsel <= _CAP - 16
                mask = (idv >= lo) & (idv < hi) & (pos >= p0) & ok
                plsc.store_compressed(
                    sel_id.at[pl.ds(nsel, 16)], idv, mask=mask)
                plsc.store_compressed(
                    sel_pos.at[pl.ds(nsel, 16)], pos, mask=mask)
                nsel = nsel + plsc.all_reduce_population_count(mask)[0]
                pnext = jnp.where(ok, pnext, jnp.minimum(pnext, gstart))
                return (nsel, pnext)

            return lax.fori_loop(0, 32, sel_group, st)

        def wait_ids(buf, sem):
            pltpu.make_async_copy(
                ids_hbm.at[pl.ds(0, 512)], buf, sem).wait()

        pltpu.async_copy(ids_hbm.at[pl.ds(0, 512)], idw0, semA)

        def sel_pair(cp, st):
            base = cp * 1024
            pltpu.async_copy(
                ids_hbm.at[pl.ds(base + 512, 512)], idw1, semB)
            wait_ids(idw0, semA)
            st = scan_ids(idw0, base, st)

            @pl.when(cp < 15)
            def _():
                pltpu.async_copy(
                    ids_hbm.at[pl.ds(base + 1024, 512)], idw0, semA)

            wait_ids(idw1, semB)
            return scan_ids(idw1, base + 512, st)

        nsel, pnext = lax.fori_loop(0, 16, sel_pair, (0, _BATCH))
        big = jnp.full((16,), jnp.int32(0x40000000))
        sel_id[pl.ds(nsel, 16)] = big
        sel_id[pl.ds(nsel + 16, 16)] = big
        ngroups = (nsel + 15) // 16

        def emit_from(ids_ref, pos_ref, ng, plo, phi, extract):
            def scan_group(g, carry):
                sid = ids_ref[pl.ds(g * 16, 16)]
                mask = (sid >= plo) & (sid < phi)
                cnt = plsc.all_reduce_population_count(mask)[0]

                @pl.when(cnt > 0)
                def _():
                    spos = pos_ref[pl.ds(g * 16, 16)]
                    plsc.store_compressed(gid.at[pl.ds(0, 16)], sid, mask=mask)
                    plsc.store_compressed(
                        gpos.at[pl.ds(0, 16)], spos, mask=mask)
                    shift = jnp.minimum(lane + 1, 15)

                    def one(t, carry2):
                        gv, pv = carry2
                        bl = gv[0] - plo
                        for k in range(2):
                            dvec = lane + (k * 16)
                            zs, ls = extract(bl, dvec)
                            rowb[pl.ds(k * 16, 16)] = zs
                            rowb[pl.ds(_DIM + k * 16, 16)] = zs + jnp.exp(ls)
                        pltpu.sync_copy(
                            rowb, shared.at[pl.ds(pv[0] * _ROW, _ROW)])
                        return (gv.at[shift].get(mode="promise_in_bounds"),
                                pv.at[shift].get(mode="promise_in_bounds"))

                    lax.fori_loop(0, cnt, one,
                                  (gid[pl.ds(0, 16)], gpos[pl.ds(0, 16)]))

                return carry

            lax.fori_loop(0, ng, scan_group, 0)

        def build_sub(plo):
            def bg(g, cnt):
                sid = sel_id[pl.ds(g * 16, 16)]
                spos = sel_pos[pl.ds(g * 16, 16)]
                ok = cnt <= _SCAP - 16
                m = (sid >= plo) & (sid < plo + 2 * _PBOX) & ok
                plsc.store_compressed(sub_id.at[pl.ds(cnt, 16)], sid, mask=m)
                plsc.store_compressed(
                    sub_pos.at[pl.ds(cnt, 16)], spos, mask=m)
                return cnt + plsc.all_reduce_population_count(m)[0]

            cnt = lax.fori_loop(0, ngroups, bg, 0)
            big = jnp.full((16,), jnp.int32(0x40000000))
            sub_id[pl.ds(cnt, 16)] = big
            sub_id[pl.ds(cnt + 16, 16)] = big
            return cnt

        def emit_rows_sub(subcnt, plo, phi, extract):
            subok = subcnt <= _SCAP - 16

            @pl.when(subok)
            def _():
                emit_from(sub_id, sub_pos, (subcnt + 15) // 16,
                          plo, phi, extract)

            @pl.when(jnp.logical_not(subok))
            def _():
                emit_from(sel_id, sel_pos, ngroups, plo, phi, extract)

        def fire(plo, zb, lb, sem):
            pltpu.async_copy(
                zt.at[c, pl.ds(0, _DIM), pl.ds(plo, _PBOX)], zb, sem)
            pltpu.async_copy(
                ldt.at[c, pl.ds(0, _DIM), pl.ds(plo, _PBOX)], lb, sem)

        def wait_piece(zb, lb, sem):
            pltpu.make_async_copy(
                zt.at[c, pl.ds(0, _DIM), pl.ds(0, _PBOX)], zb, sem).wait()
            pltpu.make_async_copy(
                ldt.at[c, pl.ds(0, _DIM), pl.ds(0, _PBOX)], lb, sem).wait()

        def process(zb, lb, plo, subcnt):
            def extract(bl, dvec):
                bvec = jnp.full((16,), 0, jnp.int32) + bl
                return (plsc.load_gather(zb, [dvec, bvec]),
                        plsc.load_gather(lb, [dvec, bvec]))

            emit_rows_sub(subcnt, plo, plo + _PBOX, extract)

        fire(lo, zst0, ldst0, semA)

        def piece2(p2, carry):
            plo = lo + p2 * (2 * _PBOX)
            fire(plo + _PBOX, zst1, ldst1, semB)
            subcnt = build_sub(plo)
            wait_piece(zst0, ldst0, semA)
            process(zst0, ldst0, plo, subcnt)

            @pl.when(p2 < _NPIECE // 2 - 1)
            def _():
                fire(plo + 2 * _PBOX, zst0, ldst0, semA)

            wait_piece(zst1, ldst1, semB)
            process(zst1, ldst1, plo + _PBOX, subcnt)
            return carry

        lax.fori_loop(0, _NPIECE // 2, piece2, 0)

        @pl.when(s == _NS - 1)
        def _():
            for tp in range(_NTSUB):
                toff = c * (_TAIL * _DIM) + tp * (_TSUB * _DIM)
                pltpu.sync_copy(ztail.at[pl.ds(toff, _TSUB * _DIM)], tailz)
                pltpu.sync_copy(ldtail.at[pl.ds(toff, _TSUB * _DIM)], taill)
                tlo = _MAIN + tp * _TSUB

                def extract(bl, dvec):
                    idx = jnp.full((16,), 0, jnp.int32) + bl * _DIM + dvec
                    return (plsc.load_gather(tailz, [idx]),
                            plsc.load_gather(taill, [idx]))

                emit_from(sel_id, sel_pos, ngroups, tlo, tlo + _TSUB, extract)

        return pnext

    p1 = do_round(0)

    @pl.when(p1 < _BATCH)
    def _():
        do_round(p1)

    plsc.subcore_barrier()
    span = _BATCH * _ROW // _NS
    obase = c * (_BATCH * _ROW)

    def wait_out(buf, sem):
        pltpu.make_async_copy(buf, out_hbm.at[pl.ds(0, _DRAIN)], sem).wait()

    def drain_pair(d, carry):
        off = s * span + d * (2 * _DRAIN)

        @pl.when(d > 0)
        def _():
            wait_out(dbounce0, semA)

        pltpu.sync_copy(shared.at[pl.ds(off, _DRAIN)], dbounce0)
        pltpu.async_copy(dbounce0, out_hbm.at[pl.ds(obase + off, _DRAIN)],
                         semA)

        @pl.when(d > 0)
        def _():
            wait_out(dbounce1, semB)

        pltpu.sync_copy(shared.at[pl.ds(off + _DRAIN, _DRAIN)], dbounce1)
        pltpu.async_copy(
            dbounce1, out_hbm.at[pl.ds(obase + off + _DRAIN, _DRAIN)], semB)
        return carry

    lax.fori_loop(0, span // (2 * _DRAIN), drain_pair, 0)
    wait_out(dbounce0, semA)
    wait_out(dbounce1, semB)


def kernel(ids, z, logdelta):
    zt = jnp.transpose(z, (0, 2, 1))
    ldt = jnp.transpose(logdelta, (0, 2, 1))
    ztail = z[:, _MAIN:, :].reshape(-1)
    ldtail = logdelta[:, _MAIN:, :].reshape(-1)
    flat = _deltaboxes_sc(ids.astype(jnp.int32), zt, ldt, ztail, ldtail)
    return flat.reshape(_NUM_MODELS, _BATCH, 2, _DIM)
